# Initial kernel scaffold; baseline (speedup 1.0000x reference)
import jax
import jax.numpy as jnp
from jax.experimental import pallas as pl


def kernel(x, edge_index, W1l, b1, W1r, g1, be1, m1, v1, W2l, b2, W2r, g2, be2, m2, v2, Wfc, bfc):
    raise NotImplementedError("write your pallas kernel here")


# speedup vs baseline: 1.9350x; 1.9350x over previous
import functools

import jax
import jax.numpy as jnp
from jax import lax
from jax.experimental import pallas as pl
from jax.experimental.pallas import tpu as pltpu
from jax.experimental.pallas import tpu_sc as plsc

N = 10000
NP = 10240
E = 160000
EB = 128
NW = 32
NBATCH = 40
EP = NW * NBATCH * EB
TRASH = N
RPW = NP // 16
ROWB = 512
EPS = 1e-5


def _sc_agg_body(nchunks, with_cnt, *refs):
    i = 0
    chunks = refs[i:i + nchunks]; i += nchunks
    src2 = refs[i]; dst2 = refs[i + 1]; i += 2
    out_p = refs[i]; i += 1
    if with_cnt:
        out_cnt = refs[i]; i += 1
    srcbuf = refs[i]; dstbuf = refs[i + 1]; rows = refs[i + 2]
    zbuf = refs[i + 3]; i += 4
    if with_cnt:
        cntbuf = refs[i]; i += 1
    acc = refs[i]; sem = refs[i + 1]

    c = lax.axis_index("c")
    s = lax.axis_index("s")
    w = c * 16 + s

    zvec = jnp.zeros((16,), jnp.float32)
    for r in range(16):
        for j in range(8):
            zbuf[r, pl.ds(j * 16, 16)] = zvec

    pltpu.sync_copy(src2.at[pl.ds(w * NBATCH, NBATCH)], srcbuf)
    pltpu.sync_copy(dst2.at[pl.ds(w * NBATCH, NBATCH)], dstbuf)

    if with_cnt:
        for r in range(NP // 16):
            cntbuf[pl.ds(r * 16, 16)] = zvec
        ones = jnp.ones((16,), jnp.float32)
        for b in range(NBATCH):
            for j in range(8):
                idx = dstbuf[b, pl.ds(j * 16, 16)]
                plsc.addupdate_scatter(cntbuf, [idx], ones)
        pltpu.sync_copy(cntbuf, out_cnt.at[w])

    for k in range(nchunks):
        for t in range(RPW // 16):
            pltpu.sync_copy(zbuf, acc.at[pl.ds(s * RPW + t * 16, 16)])
        plsc.subcore_barrier()

        def bbody(b, carry):
            pltpu.async_copy(chunks[k].at[srcbuf.at[b]], rows, sem).wait()
            pltpu.sync_copy(rows, acc.at[dstbuf.at[b]], add=True)
            return carry

        lax.fori_loop(0, NBATCH, bbody, 0)
        plsc.subcore_barrier()
        pltpu.sync_copy(acc.at[pl.ds(s * RPW, RPW)],
                        out_p.at[c, k, pl.ds(s * RPW, RPW)])
        plsc.subcore_barrier()


def _make_sc_agg(nchunks, with_cnt):
    mesh = plsc.VectorSubcoreMesh(core_axis_name="c", subcore_axis_name="s",
                                  num_cores=2, num_subcores=16)
    out_type = [jax.ShapeDtypeStruct((2, nchunks, NP, 128), jnp.float32)]
    if with_cnt:
        out_type.append(jax.ShapeDtypeStruct((NW, NP), jnp.float32))
    scratch = [
        pltpu.VMEM((NBATCH, EB), jnp.int32),
        pltpu.VMEM((NBATCH, EB), jnp.int32),
        pltpu.VMEM((EB, 128), jnp.float32),
        pltpu.VMEM((16, 128), jnp.float32),
    ]
    if with_cnt:
        scratch.append(pltpu.VMEM((NP,), jnp.float32))
    scratch += [
        pltpu.VMEM_SHARED((NP, 128), jnp.float32),
        pltpu.SemaphoreType.DMA,
    ]
    return pl.kernel(
        functools.partial(_sc_agg_body, nchunks, with_cnt),
        out_type=out_type, mesh=mesh, scratch_types=scratch,
        compiler_params=pltpu.CompilerParams(needs_layout_passes=False),
        name=f"sc_agg{nchunks}",
    )


_sc_agg2 = _make_sc_agg(2, True)
_sc_agg4 = _make_sc_agg(4, False)


def _dotT(a, w):
    return lax.dot_general(a, w, (((1,), (1,)), ((), ())),
                           preferred_element_type=jnp.float32)


def _tc1_body(p_ref, cnt_ref, x_ref, W1l_ref, b1_ref, W1r_ref,
              g1_ref, be1_ref, m1_ref, v1_ref,
              c0_ref, c1_ref, c2_ref, c3_ref, invc_ref):
    cnt = jnp.sum(cnt_ref[...], axis=0)
    invc = 1.0 / jnp.maximum(cnt, 1.0)
    agg = jnp.concatenate(
        [p_ref[0, 0] + p_ref[1, 0], p_ref[0, 1] + p_ref[1, 1]], axis=1)
    agg = agg * invc[:, None]
    h = _dotT(agg, W1l_ref[...]) + _dotT(x_ref[...], W1r_ref[...])
    h = h + b1_ref[...][None, :]
    sc = g1_ref[...] * lax.rsqrt(v1_ref[...] + EPS)
    h = (h - m1_ref[...][None, :]) * sc[None, :] + be1_ref[...][None, :]
    h = jnp.maximum(h, 0.0)
    c0_ref[...] = h[:, 0:128]
    c1_ref[...] = h[:, 128:256]
    c2_ref[...] = h[:, 256:384]
    c3_ref[...] = h[:, 384:512]
    invc_ref[...] = invc


def _tc1(p, cnt, xp, W1l, b1, W1r, g1, be1, m1, v1):
    grid = (NP // ROWB,)
    full = lambda shape: pl.BlockSpec(shape, lambda i: (0,) * len(shape))
    return pl.pallas_call(
        _tc1_body,
        grid=grid,
        in_specs=[
            pl.BlockSpec((2, 2, ROWB, 128), lambda i: (0, 0, i, 0)),
            pl.BlockSpec((NW, ROWB), lambda i: (0, i)),
            pl.BlockSpec((ROWB, 256), lambda i: (i, 0)),
            full((512, 256)), full((512,)), full((512, 256)),
            full((512,)), full((512,)), full((512,)), full((512,)),
        ],
        out_specs=[
            pl.BlockSpec((ROWB, 128), lambda i: (i, 0)),
            pl.BlockSpec((ROWB, 128), lambda i: (i, 0)),
            pl.BlockSpec((ROWB, 128), lambda i: (i, 0)),
            pl.BlockSpec((ROWB, 128), lambda i: (i, 0)),
            pl.BlockSpec((ROWB,), lambda i: (i,)),
        ],
        out_shape=[
            jax.ShapeDtypeStruct((NP, 128), jnp.float32),
            jax.ShapeDtypeStruct((NP, 128), jnp.float32),
            jax.ShapeDtypeStruct((NP, 128), jnp.float32),
            jax.ShapeDtypeStruct((NP, 128), jnp.float32),
            jax.ShapeDtypeStruct((NP,), jnp.float32),
        ],
    )(p, cnt, xp, W1l, b1, W1r, g1, be1, m1, v1)


def _tc2_body(q_ref, invc_ref, c0_ref, c1_ref, c2_ref, c3_ref,
              W2l_ref, b2_ref, W2r_ref, g2_ref, be2_ref, m2_ref, v2_ref,
              Wfc_ref, bfc_ref, out_ref):
    invc = invc_ref[...]
    agg = jnp.concatenate([q_ref[0, k] + q_ref[1, k] for k in range(4)],
                          axis=1)
    agg = agg * invc[:, None]
    hr = jnp.concatenate(
        [c0_ref[...], c1_ref[...], c2_ref[...], c3_ref[...]], axis=1)
    h = _dotT(agg, W2l_ref[...]) + _dotT(hr, W2r_ref[...])
    h = h + b2_ref[...][None, :]
    sc = g2_ref[...] * lax.rsqrt(v2_ref[...] + EPS)
    h = (h - m2_ref[...][None, :]) * sc[None, :] + be2_ref[...][None, :]
    h = jnp.maximum(h, 0.0)
    logits = _dotT(h, Wfc_ref[...]) + bfc_ref[...][None, :]
    m = jnp.max(logits, axis=1, keepdims=True)
    lse = m + jnp.log(jnp.sum(jnp.exp(logits - m), axis=1, keepdims=True))
    out_ref[...] = logits - lse


def _tc2(q, invc, c0, c1, c2, c3, W2l, b2, W2r, g2, be2, m2, v2, Wfc, bfc):
    grid = (NP // ROWB,)
    full = lambda shape: pl.BlockSpec(shape, lambda i: (0,) * len(shape))
    blk = pl.BlockSpec((ROWB, 128), lambda i: (i, 0))
    return pl.pallas_call(
        _tc2_body,
        grid=grid,
        in_specs=[
            pl.BlockSpec((2, 4, ROWB, 128), lambda i: (0, 0, i, 0)),
            pl.BlockSpec((ROWB,), lambda i: (i,)),
            blk, blk, blk, blk,
            full((512, 512)), full((512,)), full((512, 512)),
            full((512,)), full((512,)), full((512,)), full((512,)),
            full((40, 512)), full((40,)),
        ],
        out_specs=pl.BlockSpec((ROWB, 40), lambda i: (i, 0)),
        out_shape=jax.ShapeDtypeStruct((N, 40), jnp.float32),
    )(q, invc, c0, c1, c2, c3, W2l, b2, W2r, g2, be2, m2, v2, Wfc, bfc)


def kernel(x, edge_index, W1l, b1, W1r, g1, be1, m1, v1,
           W2l, b2, W2r, g2, be2, m2, v2, Wfc, bfc):
    src = edge_index[0].astype(jnp.int32)
    dst = edge_index[1].astype(jnp.int32)
    src_p = jnp.concatenate(
        [src, jnp.zeros((EP - E,), jnp.int32)]).reshape(EP // EB, EB)
    dst_p = jnp.concatenate(
        [dst, jnp.full((EP - E,), TRASH, jnp.int32)]).reshape(EP // EB, EB)
    xp = jnp.pad(x, ((0, NP - N), (0, 0)))
    x0 = xp[:, 0:128]
    x1 = xp[:, 128:256]

    p, cnt = _sc_agg2(x0, x1, src_p, dst_p)
    c0, c1, c2, c3, invc = _tc1(p, cnt, xp, W1l, b1, W1r, g1, be1, m1, v1)
    (q,) = _sc_agg4(c0, c1, c2, c3, src_p, dst_p)
    return _tc2(q, invc, c0, c1, c2, c3,
                W2l, b2, W2r, g2, be2, m2, v2, Wfc, bfc)


# speedup vs baseline: 2.0859x; 1.0780x over previous
import functools

import jax
import jax.numpy as jnp
from jax import lax
from jax.experimental import pallas as pl
from jax.experimental.pallas import tpu as pltpu
from jax.experimental.pallas import tpu_sc as plsc

N = 10000
NP = 10240
E = 160000
EB = 128
NW = 32
NBATCH = 40
EP = NW * NBATCH * EB
TRASH = N
RPW = NP // 16
ROWB = 512
EPS = 1e-5


def _sc_agg_body(nchunks, with_cnt, *refs):
    del with_cnt
    i = 0
    chunks = refs[i:i + nchunks]; i += nchunks
    src2 = refs[i]; dst2 = refs[i + 1]; i += 2
    out_p = refs[i]; i += 1
    srcbuf = refs[i]; dstbuf = refs[i + 1]
    rows0 = refs[i + 2]; rows1 = refs[i + 3]
    zbuf = refs[i + 4]; i += 5
    acc = refs[i]; sem0 = refs[i + 1]; sem1 = refs[i + 2]

    c = lax.axis_index("c")
    s = lax.axis_index("s")
    w = c * 16 + s

    zvec = jnp.zeros((16,), jnp.float32)
    for r in range(16):
        for j in range(8):
            zbuf[r, pl.ds(j * 16, 16)] = zvec

    pltpu.sync_copy(src2.at[pl.ds(w * NBATCH, NBATCH)], srcbuf)
    pltpu.sync_copy(dst2.at[pl.ds(w * NBATCH, NBATCH)], dstbuf)

    for k in range(nchunks):
        for t in range(RPW // 16):
            pltpu.sync_copy(zbuf, acc.at[pl.ds(s * RPW + t * 16, 16)])
        plsc.subcore_barrier()

        ch = chunks[k]
        pltpu.async_copy(ch.at[srcbuf.at[0]], rows0, sem0)

        @pl.loop(0, NBATCH, step=2)
        def bbody(g):
            pltpu.async_copy(ch.at[srcbuf.at[g + 1]], rows1, sem1)
            pltpu.make_async_copy(ch.at[srcbuf.at[g]], rows0, sem0).wait()
            pltpu.sync_copy(rows0, acc.at[dstbuf.at[g]], add=True)
            nxt = jnp.minimum(g + 2, NBATCH - 1)
            pltpu.async_copy(ch.at[srcbuf.at[nxt]], rows0, sem0)
            pltpu.make_async_copy(ch.at[srcbuf.at[g]], rows1, sem1).wait()
            pltpu.sync_copy(rows1, acc.at[dstbuf.at[g + 1]], add=True)

        pltpu.make_async_copy(ch.at[srcbuf.at[0]], rows0, sem0).wait()
        plsc.subcore_barrier()
        pltpu.sync_copy(acc.at[pl.ds(s * RPW, RPW)],
                        out_p.at[c, k, pl.ds(s * RPW, RPW)])
        plsc.subcore_barrier()


def _make_sc_agg(nchunks, with_cnt):
    mesh = plsc.VectorSubcoreMesh(core_axis_name="c", subcore_axis_name="s",
                                  num_cores=2, num_subcores=16)
    out_type = [jax.ShapeDtypeStruct((2, nchunks, NP, 128), jnp.float32)]
    scratch = [
        pltpu.VMEM((NBATCH, EB), jnp.int32),
        pltpu.VMEM((NBATCH, EB), jnp.int32),
        pltpu.VMEM((EB, 128), jnp.float32),
        pltpu.VMEM((EB, 128), jnp.float32),
        pltpu.VMEM((16, 128), jnp.float32),
    ]
    scratch += [
        pltpu.VMEM_SHARED((NP, 128), jnp.float32),
        pltpu.SemaphoreType.DMA,
        pltpu.SemaphoreType.DMA,
    ]
    return pl.kernel(
        functools.partial(_sc_agg_body, nchunks, with_cnt),
        out_type=out_type, mesh=mesh, scratch_types=scratch,
        compiler_params=pltpu.CompilerParams(needs_layout_passes=False),
        name=f"sc_agg{nchunks}",
    )


_sc_agg2 = _make_sc_agg(2, False)
_sc_agg4 = _make_sc_agg(4, False)


def _sc_cnt_body(dst2, out_cnt, dstbuf, cntbuf):
    c = lax.axis_index("c")
    s = lax.axis_index("s")
    w = c * 16 + s
    pltpu.sync_copy(dst2.at[pl.ds(w * NBATCH, NBATCH)], dstbuf)
    zvec = jnp.zeros((16,), jnp.float32)
    for r in range(NP // 16):
        cntbuf[pl.ds(r * 16, 16)] = zvec
    ones = jnp.ones((16,), jnp.float32)
    for b in range(NBATCH):
        for j in range(8):
            idx = dstbuf[b, pl.ds(j * 16, 16)]
            plsc.addupdate_scatter(cntbuf, [idx], ones)
    pltpu.sync_copy(cntbuf, out_cnt.at[w])


_sc_cnt = pl.kernel(
    _sc_cnt_body,
    out_type=[jax.ShapeDtypeStruct((NW, NP), jnp.float32)],
    mesh=plsc.VectorSubcoreMesh(core_axis_name="c", subcore_axis_name="s",
                                num_cores=2, num_subcores=16),
    scratch_types=[
        pltpu.VMEM((NBATCH, EB), jnp.int32),
        pltpu.VMEM((NP,), jnp.float32),
    ],
    compiler_params=pltpu.CompilerParams(needs_layout_passes=False),
    name="sc_cnt",
)


def _dotT(a, w):
    return lax.dot_general(a, w, (((1,), (1,)), ((), ())),
                           preferred_element_type=jnp.float32)


def _tc1_body(p_ref, cnt_ref, x_ref, W1l_ref, b1_ref, W1r_ref,
              g1_ref, be1_ref, m1_ref, v1_ref,
              c0_ref, c1_ref, c2_ref, c3_ref, invc_ref):
    cnt = jnp.sum(cnt_ref[...], axis=0)
    invc = 1.0 / jnp.maximum(cnt, 1.0)
    agg = jnp.concatenate(
        [p_ref[0, 0] + p_ref[1, 0], p_ref[0, 1] + p_ref[1, 1]], axis=1)
    agg = agg * invc[:, None]
    h = _dotT(agg, W1l_ref[...]) + _dotT(x_ref[...], W1r_ref[...])
    h = h + b1_ref[...][None, :]
    sc = g1_ref[...] * lax.rsqrt(v1_ref[...] + EPS)
    h = (h - m1_ref[...][None, :]) * sc[None, :] + be1_ref[...][None, :]
    h = jnp.maximum(h, 0.0)
    c0_ref[...] = h[:, 0:128]
    c1_ref[...] = h[:, 128:256]
    c2_ref[...] = h[:, 256:384]
    c3_ref[...] = h[:, 384:512]
    invc_ref[...] = invc


def _tc1(p, cnt, xp, W1l, b1, W1r, g1, be1, m1, v1):
    grid = (NP // ROWB,)
    full = lambda shape: pl.BlockSpec(shape, lambda i: (0,) * len(shape))
    return pl.pallas_call(
        _tc1_body,
        grid=grid,
        in_specs=[
            pl.BlockSpec((2, 2, ROWB, 128), lambda i: (0, 0, i, 0)),
            pl.BlockSpec((NW, ROWB), lambda i: (0, i)),
            pl.BlockSpec((ROWB, 256), lambda i: (i, 0)),
            full((512, 256)), full((512,)), full((512, 256)),
            full((512,)), full((512,)), full((512,)), full((512,)),
        ],
        out_specs=[
            pl.BlockSpec((ROWB, 128), lambda i: (i, 0)),
            pl.BlockSpec((ROWB, 128), lambda i: (i, 0)),
            pl.BlockSpec((ROWB, 128), lambda i: (i, 0)),
            pl.BlockSpec((ROWB, 128), lambda i: (i, 0)),
            pl.BlockSpec((ROWB,), lambda i: (i,)),
        ],
        out_shape=[
            jax.ShapeDtypeStruct((NP, 128), jnp.float32),
            jax.ShapeDtypeStruct((NP, 128), jnp.float32),
            jax.ShapeDtypeStruct((NP, 128), jnp.float32),
            jax.ShapeDtypeStruct((NP, 128), jnp.float32),
            jax.ShapeDtypeStruct((NP,), jnp.float32),
        ],
    )(p, cnt, xp, W1l, b1, W1r, g1, be1, m1, v1)


def _tc2_body(q_ref, invc_ref, c0_ref, c1_ref, c2_ref, c3_ref,
              W2l_ref, b2_ref, W2r_ref, g2_ref, be2_ref, m2_ref, v2_ref,
              Wfc_ref, bfc_ref, out_ref):
    invc = invc_ref[...]
    agg = jnp.concatenate([q_ref[0, k] + q_ref[1, k] for k in range(4)],
                          axis=1)
    agg = agg * invc[:, None]
    hr = jnp.concatenate(
        [c0_ref[...], c1_ref[...], c2_ref[...], c3_ref[...]], axis=1)
    h = _dotT(agg, W2l_ref[...]) + _dotT(hr, W2r_ref[...])
    h = h + b2_ref[...][None, :]
    sc = g2_ref[...] * lax.rsqrt(v2_ref[...] + EPS)
    h = (h - m2_ref[...][None, :]) * sc[None, :] + be2_ref[...][None, :]
    h = jnp.maximum(h, 0.0)
    logits = _dotT(h, Wfc_ref[...]) + bfc_ref[...][None, :]
    m = jnp.max(logits, axis=1, keepdims=True)
    lse = m + jnp.log(jnp.sum(jnp.exp(logits - m), axis=1, keepdims=True))
    out_ref[...] = logits - lse


def _tc2(q, invc, c0, c1, c2, c3, W2l, b2, W2r, g2, be2, m2, v2, Wfc, bfc):
    grid = (NP // ROWB,)
    full = lambda shape: pl.BlockSpec(shape, lambda i: (0,) * len(shape))
    blk = pl.BlockSpec((ROWB, 128), lambda i: (i, 0))
    return pl.pallas_call(
        _tc2_body,
        grid=grid,
        in_specs=[
            pl.BlockSpec((2, 4, ROWB, 128), lambda i: (0, 0, i, 0)),
            pl.BlockSpec((ROWB,), lambda i: (i,)),
            blk, blk, blk, blk,
            full((512, 512)), full((512,)), full((512, 512)),
            full((512,)), full((512,)), full((512,)), full((512,)),
            full((40, 512)), full((40,)),
        ],
        out_specs=pl.BlockSpec((ROWB, 40), lambda i: (i, 0)),
        out_shape=jax.ShapeDtypeStruct((N, 40), jnp.float32),
    )(q, invc, c0, c1, c2, c3, W2l, b2, W2r, g2, be2, m2, v2, Wfc, bfc)


def kernel(x, edge_index, W1l, b1, W1r, g1, be1, m1, v1,
           W2l, b2, W2r, g2, be2, m2, v2, Wfc, bfc):
    src = edge_index[0].astype(jnp.int32)
    dst = edge_index[1].astype(jnp.int32)
    src_p = jnp.concatenate(
        [src, jnp.zeros((EP - E,), jnp.int32)]).reshape(EP // EB, EB)
    dst_p = jnp.concatenate(
        [dst, jnp.full((EP - E,), TRASH, jnp.int32)]).reshape(EP // EB, EB)
    xp = jnp.pad(x, ((0, NP - N), (0, 0)))
    x0 = xp[:, 0:128]
    x1 = xp[:, 128:256]

    (cnt,) = _sc_cnt(dst_p)
    (p,) = _sc_agg2(x0, x1, src_p, dst_p)
    c0, c1, c2, c3, invc = _tc1(p, cnt, xp, W1l, b1, W1r, g1, be1, m1, v1)
    (q,) = _sc_agg4(c0, c1, c2, c3, src_p, dst_p)
    return _tc2(q, invc, c0, c1, c2, c3,
                W2l, b2, W2r, g2, be2, m2, v2, Wfc, bfc)
